# Initial kernel scaffold; baseline (speedup 1.0000x reference)
import jax
import jax.numpy as jnp
from jax.experimental import pallas as pl


def kernel(x, atten):
    raise NotImplementedError("write your pallas kernel here")


# speedup vs baseline: 1.6233x; 1.6233x over previous
import functools

import jax
import jax.numpy as jnp
from jax.experimental import pallas as pl
from jax.experimental.pallas import tpu as pltpu

H = 12
S = 2048
D = 768
B = 2
OUT_LEN = 512
N_CLUSTER = 5
TOP_K = OUT_LEN - 1 - N_CLUSTER
N_TAIL = (S - 1) - TOP_K - 1
CLUSTER_LEN = (N_TAIL + N_CLUSTER - 1) // N_CLUSTER * 1
PAD = N_CLUSTER - N_TAIL % N_CLUSTER
CLUSTER_ROWS = (N_TAIL + PAD) // N_CLUSTER

ROW_CHUNK = 512
N_RC = S // ROW_CHUNK


def _reduce_kernel(a_ref, av_ref, acc_ref, comp_ref, dacc_ref):
    h = pl.program_id(1)
    rc = pl.program_id(2)
    blk = a_ref[0]
    part = jnp.sum(blk, axis=0, keepdims=True) * (1.0 / H)
    r0 = rc * ROW_CHUNK
    ri = jax.lax.broadcasted_iota(jnp.int32, (ROW_CHUNK, S), 0) + r0
    ci = jax.lax.broadcasted_iota(jnp.int32, (ROW_CHUNK, S), 1)
    dpart = jnp.sum(jnp.where(ri == ci, blk, 0.0), axis=0, keepdims=True) * (1.0 / H)

    @pl.when(jnp.logical_and(h == 0, rc == 0))
    def _():
        acc_ref[...] = jnp.zeros_like(acc_ref)
        comp_ref[...] = jnp.zeros_like(comp_ref)
        dacc_ref[...] = jnp.zeros_like(dacc_ref)

    y = part - comp_ref[...]
    t = acc_ref[...] + y
    comp_ref[...] = (t - acc_ref[...]) - y
    acc_ref[...] = t
    dacc_ref[...] = dacc_ref[...] + dpart

    @pl.when(jnp.logical_and(h == H - 1, rc == N_RC - 1))
    def _():
        av_ref[0] = acc_ref[...] - dacc_ref[...]


CH = 256
N_CH = S // CH


def _select_kernel(av_ref, x_ref, o_ref):
    pos = jax.lax.broadcasted_iota(jnp.int32, (1, S), 1)
    v = jnp.where(pos == 0, -1e30, av_ref[0])
    vcol = jnp.transpose(v)

    rank_chunks = []
    for c in range(N_CH):
        vp = v[:, c * CH:(c + 1) * CH]
        qi = jax.lax.broadcasted_iota(jnp.int32, (S, CH), 0)
        pi = jax.lax.broadcasted_iota(jnp.int32, (S, CH), 1) + c * CH
        beats = (vcol > vp) | ((vcol == vp) & (qi < pi))
        rank_chunks.append(jnp.sum(jnp.where(beats, 1.0, 0.0), axis=0, keepdims=True))
    rank = jnp.concatenate(rank_chunks, axis=1)

    valid = pos > 0
    top = valid & (rank < float(TOP_K))
    tail = valid & (rank > float(TOP_K))

    topcol = jnp.transpose(jnp.where(top, 1.0, 0.0))
    tailcol = jnp.transpose(jnp.where(tail, 1.0, 0.0))
    ptop_chunks = []
    ptail_chunks = []
    for c in range(N_CH):
        qi = jax.lax.broadcasted_iota(jnp.int32, (S, CH), 0)
        pi = jax.lax.broadcasted_iota(jnp.int32, (S, CH), 1) + c * CH
        lower = jnp.where(qi < pi, 1.0, 0.0)
        ptop_chunks.append(jnp.sum(lower * topcol, axis=0, keepdims=True))
        ptail_chunks.append(jnp.sum(lower * tailcol, axis=0, keepdims=True))
    p_top = jnp.concatenate(ptop_chunks, axis=1)
    p_tail = jnp.concatenate(ptail_chunks, axis=1)

    neg = jnp.float32(-1e30)
    m = jnp.max(jnp.where(tail, v, neg))
    e = jnp.where(tail, jnp.exp(v - m), 0.0)
    w = e / jnp.sum(e)

    cluster = jnp.zeros_like(p_tail)
    for k in range(1, N_CLUSTER):
        cluster = cluster + jnp.where(p_tail >= float(k * CLUSTER_ROWS), 1.0, 0.0)

    row_f = jnp.where(top, 1.0 + p_top,
                      jnp.where(tail, float(OUT_LEN - N_CLUSTER) + cluster, -1.0))
    row_f = jnp.where(pos == 0, 0.0, row_f)
    row = row_f.astype(jnp.int32)
    val = jnp.where(top, 1.0,
                    jnp.where(tail, w * (1.0 / CLUSTER_ROWS), 0.0))
    val = jnp.where(pos == 0, 1.0, val)

    oi = jax.lax.broadcasted_iota(jnp.int32, (OUT_LEN, S), 0)
    sel = jnp.where(oi == row, val, 0.0)
    xb = x_ref[0]
    o_ref[0] = jax.lax.dot_general(
        sel, xb, (((1,), (0,)), ((), ())),
        precision=jax.lax.Precision.HIGHEST,
        preferred_element_type=jnp.float32)


@functools.partial(jax.jit, static_argnames=())
def kernel(x, atten):
    av = pl.pallas_call(
        _reduce_kernel,
        grid=(B, H, N_RC),
        in_specs=[pl.BlockSpec((1, ROW_CHUNK, S),
                               lambda b, h, rc: (b * H + h, rc, 0))],
        out_specs=pl.BlockSpec((1, 1, S), lambda b, h, rc: (b, 0, 0)),
        out_shape=jax.ShapeDtypeStruct((B, 1, S), jnp.float32),
        scratch_shapes=[
            pltpu.VMEM((1, S), jnp.float32),
            pltpu.VMEM((1, S), jnp.float32),
            pltpu.VMEM((1, S), jnp.float32),
        ],
    )(atten)

    out = pl.pallas_call(
        _select_kernel,
        grid=(B,),
        in_specs=[
            pl.BlockSpec((1, 1, S), lambda b: (b, 0, 0)),
            pl.BlockSpec((1, S, D), lambda b: (b, 0, 0)),
        ],
        out_specs=pl.BlockSpec((1, OUT_LEN, D), lambda b: (b, 0, 0)),
        out_shape=jax.ShapeDtypeStruct((B, OUT_LEN, D), jnp.float32),
    )(av, x)
    return out
